# baseline (speedup 1.0000x reference)
import functools

import jax
import jax.numpy as jnp
from jax.experimental import pallas as pl
from jax.experimental.pallas import tpu as pltpu

N = 4096
BM = 512
BK = 600


def _bf(t):
    return t.astype(jnp.bfloat16)


def _dot0(a, b):
    return jax.lax.dot_general(a, b, (((0,), (0,)), ((), ())),
                               preferred_element_type=jnp.float32)


def _trunk_body(xt_ref, w1, b1, w2, b2, w3, b3, wh, bh, wd1, bd1, wd2, bd2,
                wd3, bd3, ht_ref, d3t_ref):
    def lin(t, w, b):
        return _dot0(w[...], t) + b[...]

    t = jnp.maximum(lin(xt_ref[...], w1, b1), 0.0)
    t = jnp.maximum(lin(t, w2, b2), 0.0)
    t = jnp.maximum(lin(t, w3, b3), 0.0)
    ht = lin(t, wh, bh)
    ht_ref[...] = ht
    t = jnp.maximum(lin(ht, wd1, bd1), 0.0)
    t = jnp.maximum(lin(t, wd2, bd2), 0.0)
    d3t = jnp.maximum(lin(t, wd3, bd3), 0.0)
    d3t_ref[...] = _bf(d3t)


def _run_trunk(xt, ws, bs):
    full = lambda a: pl.BlockSpec(a.shape, lambda i: (0,) * a.ndim)
    in_specs = [pl.BlockSpec((3000, BM), lambda i: (0, i))]
    args = []
    for w, b in zip(ws, bs):
        in_specs += [full(w), full(b)]
        args += [w, b]
    return pl.pallas_call(
        _trunk_body,
        grid=(N // BM,),
        in_specs=in_specs,
        out_specs=(pl.BlockSpec((64, BM), lambda i: (0, i)),
                   pl.BlockSpec((1024, BM), lambda i: (0, i))),
        out_shape=(jax.ShapeDtypeStruct((64, N), jnp.float32),
                   jax.ShapeDtypeStruct((1024, N), jnp.bfloat16)),
    )(xt, *args)


def _head_body(wt_ref, b_ref, d3t_ref, o_ref, *, act):
    acc = jnp.dot(_bf(wt_ref[...]), d3t_ref[...],
                  preferred_element_type=jnp.float32)
    acc = acc + b_ref[...]
    if act == "sigmoid":
        acc = jax.nn.sigmoid(acc)
    elif act == "expclip":
        acc = jnp.exp(jnp.clip(acc, -15.0, 15.0))
    elif act == "softplus":
        acc = jax.nn.softplus(acc)
    o_ref[...] = acc


def _run_head(d3t, wt, b, act):
    return pl.pallas_call(
        functools.partial(_head_body, act=act),
        grid=(3000 // BK,),
        in_specs=[pl.BlockSpec((BK, 1024), lambda i: (i, 0)),
                  pl.BlockSpec((BK, 1), lambda i: (i, 0)),
                  pl.BlockSpec(d3t.shape, lambda i: (0, 0))],
        out_specs=pl.BlockSpec((BK, N), lambda i: (i, 0)),
        out_shape=jax.ShapeDtypeStruct((3000, N), jnp.float32),
    )(wt, b, d3t)


def _gcn_body(act_ref, adj_ref, v_ref, w_ref, o_ref, u_ref, *, apply_act,
              out_bf16, v_transposed):
    @pl.when(pl.program_id(0) == 0)
    def _():
        if v_transposed:
            u = _dot0(v_ref[...], w_ref[...])
        else:
            u = jnp.dot(v_ref[...], w_ref[...],
                        preferred_element_type=jnp.float32)
        u_ref[...] = _bf(u)

    t = jnp.dot(adj_ref[...], u_ref[...], preferred_element_type=jnp.float32)
    if apply_act:
        t = jnp.where(act_ref[0] != 0, jnp.maximum(t, 0.0), t)
    o_ref[...] = _bf(t) if out_bf16 else t


def _run_gcn(active_s, adjb, v, w, apply_act, out_bf16, v_transposed=False):
    fo = w.shape[1]
    return pl.pallas_call(
        functools.partial(_gcn_body, apply_act=apply_act, out_bf16=out_bf16,
                          v_transposed=v_transposed),
        grid=(N // BM,),
        in_specs=[pl.BlockSpec(memory_space=pltpu.SMEM),
                  pl.BlockSpec((BM, N), lambda i: (i, 0)),
                  pl.BlockSpec(v.shape, lambda i: (0, 0)),
                  pl.BlockSpec(w.shape, lambda i: (0, 0))],
        out_specs=pl.BlockSpec((BM, fo), lambda i: (i, 0)),
        out_shape=jax.ShapeDtypeStruct(
            (N, fo), jnp.bfloat16 if out_bf16 else jnp.float32),
        scratch_shapes=[pltpu.VMEM((N, fo), jnp.bfloat16)],
    )(active_s, adjb, v, w)


def _adjhat_body(zr_ref, zc_ref, o_ref):
    acc = jax.lax.dot_general(zr_ref[...], zc_ref[...],
                              (((1,), (1,)), ((), ())),
                              preferred_element_type=jnp.float32)
    o_ref[...] = jax.nn.sigmoid(acc)


def _run_adjhat(z):
    bn = 2048
    return pl.pallas_call(
        _adjhat_body,
        grid=(N // BM, N // bn),
        in_specs=[pl.BlockSpec((BM, 16), lambda i, j: (i, 0)),
                  pl.BlockSpec((bn, 16), lambda i, j: (j, 0))],
        out_specs=pl.BlockSpec((BM, bn), lambda i, j: (i, j)),
        out_shape=jax.ShapeDtypeStruct((N, N), jnp.float32),
    )(z, z)


def kernel(x, adj, active, params):
    p = params
    adjb = _bf(adj)
    active_s = jnp.reshape(jnp.asarray(active, jnp.int32), (1,))

    trunk_w = [p[k] for k in
               ("W_en1", "W_en2", "W_en3", "W_h", "W_de1", "W_de2", "W_de3")]
    trunk_b = [jnp.reshape(p[k], (-1, 1)) for k in
               ("b_en1", "b_en2", "b_en3", "b_h", "b_de1", "b_de2", "b_de3")]
    ht, d3t = _run_trunk(x.T, trunk_w, trunk_b)
    h = ht.T

    x_hat = _run_head(d3t, p["W_xhat"].T, jnp.reshape(p["b_xhat"], (-1, 1)),
                      "none").T
    pi = _run_head(d3t, p["W_pi"].T, jnp.reshape(p["b_pi"], (-1, 1)),
                   "sigmoid").T
    mu = _run_head(d3t, p["W_mu"].T, jnp.reshape(p["b_mu"], (-1, 1)),
                   "expclip").T
    theta = _run_head(d3t, p["W_theta"].T, jnp.reshape(p["b_theta"], (-1, 1)),
                      "softplus").T

    g1 = _run_gcn(active_s, adjb, ht, p["Wg1"], True, True, v_transposed=True)
    g2 = _run_gcn(active_s, adjb, g1, p["Wg2"], True, True)
    z = _run_gcn(active_s, adjb, g2, p["Wgz"], False, False)
    adj_hat = _run_adjhat(z)
    dz1 = _run_gcn(active_s, adjb, z, p["Wd1"], True, True)
    dz2 = _run_gcn(active_s, adjb, dz1, p["Wd2"], True, True)
    z_hat = _run_gcn(active_s, adjb, dz2, p["Wdz"], False, False)

    return (x_hat, pi, mu, theta, z, adj_hat, z_hat, h)


# speedup vs baseline: 1.0430x; 1.0430x over previous
import functools

import jax
import jax.numpy as jnp
from jax.experimental import pallas as pl
from jax.experimental.pallas import tpu as pltpu

N = 4096
BM = 512
BK = 1000


def _bf(t):
    return t.astype(jnp.bfloat16)


def _dot0(a, b):
    return jax.lax.dot_general(a, b, (((0,), (0,)), ((), ())),
                               preferred_element_type=jnp.float32)


def _trunk_body(xt_ref, w1, b1, w2, b2, w3, b3, wh, bh, wd1, bd1, wd2, bd2,
                wd3, bd3, ht_ref, d3t_ref):
    def lin(t, w, b):
        return _dot0(w[...], t) + b[...]

    t = jnp.maximum(lin(xt_ref[...], w1, b1), 0.0)
    t = jnp.maximum(lin(t, w2, b2), 0.0)
    t = jnp.maximum(lin(t, w3, b3), 0.0)
    ht = lin(t, wh, bh)
    ht_ref[...] = ht
    t = jnp.maximum(lin(ht, wd1, bd1), 0.0)
    t = jnp.maximum(lin(t, wd2, bd2), 0.0)
    d3t = jnp.maximum(lin(t, wd3, bd3), 0.0)
    d3t_ref[...] = _bf(d3t)


def _run_trunk(xt, ws, bs):
    full = lambda a: pl.BlockSpec(a.shape, lambda i: (0,) * a.ndim)
    in_specs = [pl.BlockSpec((3000, BM), lambda i: (0, i))]
    args = []
    for w, b in zip(ws, bs):
        in_specs += [full(w), full(b)]
        args += [w, b]
    return pl.pallas_call(
        _trunk_body,
        grid=(N // BM,),
        in_specs=in_specs,
        out_specs=(pl.BlockSpec((64, BM), lambda i: (0, i)),
                   pl.BlockSpec((1024, BM), lambda i: (0, i))),
        out_shape=(jax.ShapeDtypeStruct((64, N), jnp.float32),
                   jax.ShapeDtypeStruct((1024, N), jnp.bfloat16)),
    )(xt, *args)


def _head_body(wt_ref, b_ref, d3t_ref, o_ref, *, act):
    acc = jnp.dot(_bf(wt_ref[...]), d3t_ref[...],
                  preferred_element_type=jnp.float32)
    acc = acc + b_ref[0]
    if act == "sigmoid":
        acc = jax.nn.sigmoid(acc)
    elif act == "expclip":
        acc = jnp.exp(jnp.clip(acc, -15.0, 15.0))
    elif act == "softplus":
        acc = jax.nn.softplus(acc)
    o_ref[...] = acc


def _run_head(d3t, wt, bstack, head, act):
    return pl.pallas_call(
        functools.partial(_head_body, act=act),
        grid=(3000 // BK,),
        in_specs=[pl.BlockSpec((BK, 1024), lambda i: (i, 0)),
                  pl.BlockSpec((1, BK, 1), lambda i: (head, i, 0)),
                  pl.BlockSpec(d3t.shape, lambda i: (0, 0))],
        out_specs=pl.BlockSpec((BK, N), lambda i: (i, 0)),
        out_shape=jax.ShapeDtypeStruct((3000, N), jnp.float32),
    )(wt, bstack, d3t)


def _gcn_first_body(act_ref, adj_ref, v_ref, w_ref, o_ref, adjb_ref, u_ref):
    @pl.when(pl.program_id(0) == 0)
    def _():
        u_ref[...] = _bf(_dot0(v_ref[...], w_ref[...]))

    ab = _bf(adj_ref[...])
    adjb_ref[...] = ab
    t = jnp.dot(ab, u_ref[...], preferred_element_type=jnp.float32)
    t = jnp.where(act_ref[0] != 0, jnp.maximum(t, 0.0), t)
    o_ref[...] = _bf(t)


def _run_gcn_first(active_s, adj, vt, w):
    fo = w.shape[1]
    return pl.pallas_call(
        _gcn_first_body,
        grid=(N // BM,),
        in_specs=[pl.BlockSpec(memory_space=pltpu.SMEM),
                  pl.BlockSpec((BM, N), lambda i: (i, 0)),
                  pl.BlockSpec(vt.shape, lambda i: (0, 0)),
                  pl.BlockSpec(w.shape, lambda i: (0, 0))],
        out_specs=(pl.BlockSpec((BM, fo), lambda i: (i, 0)),
                   pl.BlockSpec((BM, N), lambda i: (i, 0))),
        out_shape=(jax.ShapeDtypeStruct((N, fo), jnp.bfloat16),
                   jax.ShapeDtypeStruct((N, N), jnp.bfloat16)),
        scratch_shapes=[pltpu.VMEM((N, fo), jnp.bfloat16)],
    )(active_s, adj, vt, w)


def _gcn_body(act_ref, adj_ref, v_ref, w_ref, o_ref, u_ref, *, apply_act,
              out_bf16):
    @pl.when(pl.program_id(0) == 0)
    def _():
        u_ref[...] = _bf(jnp.dot(v_ref[...], w_ref[...],
                                 preferred_element_type=jnp.float32))

    t = jnp.dot(adj_ref[...], u_ref[...], preferred_element_type=jnp.float32)
    if apply_act:
        t = jnp.where(act_ref[0] != 0, jnp.maximum(t, 0.0), t)
    o_ref[...] = _bf(t) if out_bf16 else t


def _run_gcn(active_s, adjb, v, w, apply_act, out_bf16):
    fo = w.shape[1]
    return pl.pallas_call(
        functools.partial(_gcn_body, apply_act=apply_act, out_bf16=out_bf16),
        grid=(N // BM,),
        in_specs=[pl.BlockSpec(memory_space=pltpu.SMEM),
                  pl.BlockSpec((BM, N), lambda i: (i, 0)),
                  pl.BlockSpec(v.shape, lambda i: (0, 0)),
                  pl.BlockSpec(w.shape, lambda i: (0, 0))],
        out_specs=pl.BlockSpec((BM, fo), lambda i: (i, 0)),
        out_shape=jax.ShapeDtypeStruct(
            (N, fo), jnp.bfloat16 if out_bf16 else jnp.float32),
        scratch_shapes=[pltpu.VMEM((N, fo), jnp.bfloat16)],
    )(active_s, adjb, v, w)


def _adjhat_body(zr_ref, zc_ref, o_ref):
    acc = jax.lax.dot_general(zr_ref[...], zc_ref[...],
                              (((1,), (1,)), ((), ())),
                              preferred_element_type=jnp.float32)
    o_ref[...] = jax.nn.sigmoid(acc)


def _run_adjhat(z):
    bm, bn = 1024, 4096
    return pl.pallas_call(
        _adjhat_body,
        grid=(N // bm, N // bn),
        in_specs=[pl.BlockSpec((bm, 16), lambda i, j: (i, 0)),
                  pl.BlockSpec((bn, 16), lambda i, j: (j, 0))],
        out_specs=pl.BlockSpec((bm, bn), lambda i, j: (i, j)),
        out_shape=jax.ShapeDtypeStruct((N, N), jnp.float32),
    )(z, z)


def kernel(x, adj, active, params):
    p = params
    active_s = jnp.reshape(jnp.asarray(active, jnp.int32), (1,))

    trunk_w = [p[k] for k in
               ("W_en1", "W_en2", "W_en3", "W_h", "W_de1", "W_de2", "W_de3")]
    trunk_b = [jnp.reshape(p[k], (-1, 1)) for k in
               ("b_en1", "b_en2", "b_en3", "b_h", "b_de1", "b_de2", "b_de3")]
    ht, d3t = _run_trunk(x.T, trunk_w, trunk_b)
    h = ht.T

    bstack = jnp.reshape(
        jnp.stack([p["b_xhat"], p["b_pi"], p["b_mu"], p["b_theta"]]),
        (4, 3000, 1))
    x_hat = _run_head(d3t, p["W_xhat"].T, bstack, 0, "none").T
    pi = _run_head(d3t, p["W_pi"].T, bstack, 1, "sigmoid").T
    mu = _run_head(d3t, p["W_mu"].T, bstack, 2, "expclip").T
    theta = _run_head(d3t, p["W_theta"].T, bstack, 3, "softplus").T

    g1, adjb = _run_gcn_first(active_s, adj, ht, p["Wg1"])
    g2 = _run_gcn(active_s, adjb, g1, p["Wg2"], True, True)
    z = _run_gcn(active_s, adjb, g2, p["Wgz"], False, False)
    adj_hat = _run_adjhat(z)
    dz1 = _run_gcn(active_s, adjb, z, p["Wd1"], True, True)
    dz2 = _run_gcn(active_s, adjb, dz1, p["Wd2"], True, True)
    z_hat = _run_gcn(active_s, adjb, dz2, p["Wdz"], False, False)

    return (x_hat, pi, mu, theta, z, adj_hat, z_hat, h)


# speedup vs baseline: 1.1007x; 1.0554x over previous
import functools

import jax
import jax.numpy as jnp
from jax.experimental import pallas as pl
from jax.experimental.pallas import tpu as pltpu

N = 4096
BM = 512
BK = 600


def _bf(t):
    return t.astype(jnp.bfloat16)


def _dot0(a, b):
    return jax.lax.dot_general(a, b, (((0,), (0,)), ((), ())),
                               preferred_element_type=jnp.float32)


def _trunk_body(xt_ref, w1, b1, w2, b2, w3, b3, wh, bh, wd1, bd1, wd2, bd2,
                wd3, bd3, ht_ref, d3t_ref):
    def lin(t, w, b):
        return _dot0(w[...], t) + b[...]

    t = jnp.maximum(lin(xt_ref[...], w1, b1), 0.0)
    t = jnp.maximum(lin(t, w2, b2), 0.0)
    t = jnp.maximum(lin(t, w3, b3), 0.0)
    ht = lin(t, wh, bh)
    ht_ref[...] = ht
    t = jnp.maximum(lin(ht, wd1, bd1), 0.0)
    t = jnp.maximum(lin(t, wd2, bd2), 0.0)
    d3t = jnp.maximum(lin(t, wd3, bd3), 0.0)
    d3t_ref[...] = _bf(d3t)


def _run_trunk(xt, ws, bs):
    full = lambda a: pl.BlockSpec(a.shape, lambda i: (0,) * a.ndim)
    in_specs = [pl.BlockSpec((3000, BM), lambda i: (0, i))]
    args = []
    for w, b in zip(ws, bs):
        in_specs += [full(w), full(b)]
        args += [w, b]
    return pl.pallas_call(
        _trunk_body,
        grid=(N // BM,),
        in_specs=in_specs,
        out_specs=(pl.BlockSpec((64, BM), lambda i: (0, i)),
                   pl.BlockSpec((1024, BM), lambda i: (0, i))),
        out_shape=(jax.ShapeDtypeStruct((64, N), jnp.float32),
                   jax.ShapeDtypeStruct((1024, N), jnp.bfloat16)),
    )(xt, *args)


def _head_body(wt_ref, b_ref, d3t_ref, o_ref, *, act):
    acc = jnp.dot(_bf(wt_ref[...]), d3t_ref[...],
                  preferred_element_type=jnp.float32)
    acc = acc + b_ref[0]
    if act == "sigmoid":
        acc = jax.nn.sigmoid(acc)
    elif act == "expclip":
        acc = jnp.exp(jnp.clip(acc, -15.0, 15.0))
    elif act == "softplus":
        m = jnp.minimum(acc, 20.0) * 1.4426950408889634
        sp = 0.6931471805599453 * jnp.log2(1.0 + jnp.exp2(m))
        acc = jnp.where(acc > 20.0, acc, sp)
    o_ref[...] = acc


def _run_head(d3t, wt, bstack, head, act):
    return pl.pallas_call(
        functools.partial(_head_body, act=act),
        grid=(3000 // BK,),
        in_specs=[pl.BlockSpec((BK, 1024), lambda i: (i, 0)),
                  pl.BlockSpec((1, BK, 1), lambda i: (head, i, 0)),
                  pl.BlockSpec(d3t.shape, lambda i: (0, 0))],
        out_specs=pl.BlockSpec((BK, N), lambda i: (i, 0)),
        out_shape=jax.ShapeDtypeStruct((3000, N), jnp.float32),
    )(wt, bstack, d3t)


def _gcn_first_body(act_ref, adj_ref, v_ref, w_ref, o_ref, adjb_ref, u_ref):
    @pl.when(pl.program_id(0) == 0)
    def _():
        u_ref[...] = _bf(_dot0(v_ref[...], w_ref[...]))

    ab = _bf(adj_ref[...])
    adjb_ref[...] = ab
    t = jnp.dot(ab, u_ref[...], preferred_element_type=jnp.float32)
    t = jnp.where(act_ref[0] != 0, jnp.maximum(t, 0.0), t)
    o_ref[...] = _bf(t)


def _run_gcn_first(active_s, adj, vt, w):
    fo = w.shape[1]
    return pl.pallas_call(
        _gcn_first_body,
        grid=(N // BM,),
        in_specs=[pl.BlockSpec(memory_space=pltpu.SMEM),
                  pl.BlockSpec((BM, N), lambda i: (i, 0)),
                  pl.BlockSpec(vt.shape, lambda i: (0, 0)),
                  pl.BlockSpec(w.shape, lambda i: (0, 0))],
        out_specs=(pl.BlockSpec((BM, fo), lambda i: (i, 0)),
                   pl.BlockSpec((BM, N), lambda i: (i, 0))),
        out_shape=(jax.ShapeDtypeStruct((N, fo), jnp.bfloat16),
                   jax.ShapeDtypeStruct((N, N), jnp.bfloat16)),
        scratch_shapes=[pltpu.VMEM((N, fo), jnp.bfloat16)],
    )(active_s, adj, vt, w)


def _gcn_body(act_ref, adj_ref, v_ref, w_ref, o_ref, u_ref, *, apply_act,
              out_bf16):
    @pl.when(pl.program_id(0) == 0)
    def _():
        u_ref[...] = _bf(jnp.dot(v_ref[...], w_ref[...],
                                 preferred_element_type=jnp.float32))

    t = jnp.dot(adj_ref[...], u_ref[...], preferred_element_type=jnp.float32)
    if apply_act:
        t = jnp.where(act_ref[0] != 0, jnp.maximum(t, 0.0), t)
    o_ref[...] = _bf(t) if out_bf16 else t


def _run_gcn(active_s, adjb, v, w, apply_act, out_bf16):
    fo = w.shape[1]
    return pl.pallas_call(
        functools.partial(_gcn_body, apply_act=apply_act, out_bf16=out_bf16),
        grid=(N // BM,),
        in_specs=[pl.BlockSpec(memory_space=pltpu.SMEM),
                  pl.BlockSpec((BM, N), lambda i: (i, 0)),
                  pl.BlockSpec(v.shape, lambda i: (0, 0)),
                  pl.BlockSpec(w.shape, lambda i: (0, 0))],
        out_specs=pl.BlockSpec((BM, fo), lambda i: (i, 0)),
        out_shape=jax.ShapeDtypeStruct(
            (N, fo), jnp.bfloat16 if out_bf16 else jnp.float32),
        scratch_shapes=[pltpu.VMEM((N, fo), jnp.bfloat16)],
    )(active_s, adjb, v, w)


def _adjhat_body(zr_ref, zc_ref, o_ref):
    acc = jax.lax.dot_general(zr_ref[...], zc_ref[...],
                              (((1,), (1,)), ((), ())),
                              preferred_element_type=jnp.float32)
    o_ref[...] = jax.nn.sigmoid(acc)


def _run_adjhat(z):
    bm, bn = 1024, 4096
    return pl.pallas_call(
        _adjhat_body,
        grid=(N // bm, N // bn),
        in_specs=[pl.BlockSpec((bm, 16), lambda i, j: (i, 0)),
                  pl.BlockSpec((bn, 16), lambda i, j: (j, 0))],
        out_specs=pl.BlockSpec((bm, bn), lambda i, j: (i, j)),
        out_shape=jax.ShapeDtypeStruct((N, N), jnp.float32),
    )(z, z)


def kernel(x, adj, active, params):
    p = params
    active_s = jnp.reshape(jnp.asarray(active, jnp.int32), (1,))

    trunk_w = [p[k] for k in
               ("W_en1", "W_en2", "W_en3", "W_h", "W_de1", "W_de2", "W_de3")]
    trunk_b = [jnp.reshape(p[k], (-1, 1)) for k in
               ("b_en1", "b_en2", "b_en3", "b_h", "b_de1", "b_de2", "b_de3")]
    ht, d3t = _run_trunk(x.T, trunk_w, trunk_b)
    h = ht.T

    bstack = jnp.reshape(
        jnp.stack([p["b_xhat"], p["b_pi"], p["b_mu"], p["b_theta"]]),
        (4, 3000, 1))
    x_hat = _run_head(d3t, p["W_xhat"].T, bstack, 0, "none").T
    pi = _run_head(d3t, p["W_pi"].T, bstack, 1, "sigmoid").T
    mu = _run_head(d3t, p["W_mu"].T, bstack, 2, "expclip").T
    theta = _run_head(d3t, p["W_theta"].T, bstack, 3, "softplus").T

    g1, adjb = _run_gcn_first(active_s, adj, ht, p["Wg1"])
    g2 = _run_gcn(active_s, adjb, g1, p["Wg2"], True, True)
    z = _run_gcn(active_s, adjb, g2, p["Wgz"], False, False)
    adj_hat = _run_adjhat(z)
    dz1 = _run_gcn(active_s, adjb, z, p["Wd1"], True, True)
    dz2 = _run_gcn(active_s, adjb, dz1, p["Wd2"], True, True)
    z_hat = _run_gcn(active_s, adjb, dz2, p["Wdz"], False, False)

    return (x_hat, pi, mu, theta, z, adj_hat, z_hat, h)
